# baseline (speedup 1.0000x reference)
import functools

import jax
import jax.numpy as jnp
from jax import lax
from jax.experimental import pallas as pl

_NOISE_LEVEL = 1e-05


def _fused_kernel(state_ref, w_ref, noise_ref, out_ref):
    q = jnp.dot(state_ref[...], w_ref[...], preferred_element_type=jnp.float32)
    q = q + noise_ref[...]
    j = jnp.argmax(q, axis=-1)
    a = q.shape[-1]
    iota = lax.broadcasted_iota(jnp.int32, q.shape, 1)
    out_ref[...] = (iota == j[:, None]).astype(jnp.float32)


@functools.partial(jax.jit, static_argnames=("tb",))
def _run(state, W, noise, tb):
    B, d = state.shape
    A = W.shape[1]
    grid = (B // tb,)
    return pl.pallas_call(
        _fused_kernel,
        grid=grid,
        in_specs=[
            pl.BlockSpec((tb, d), lambda i: (i, 0)),
            pl.BlockSpec((d, A), lambda i: (0, 0)),
            pl.BlockSpec((tb, A), lambda i: (i, 0)),
        ],
        out_specs=pl.BlockSpec((tb, A), lambda i: (i, 0)),
        out_shape=jax.ShapeDtypeStruct((B, A), jnp.float32),
    )(state, W, noise)


def kernel(state, W):
    B, d = state.shape
    A = W.shape[1]
    rand = jax.random.uniform(jax.random.key(1), (B, A), dtype=jnp.float32)
    noise = (rand * 2 - 1) * _NOISE_LEVEL
    tb = 256 if B % 256 == 0 else B
    return _run(state, W, noise, tb)


# speedup vs baseline: 1.8169x; 1.8169x over previous
import functools

import jax
import jax.numpy as jnp
from jax import lax
from jax.experimental import pallas as pl

_NOISE_LEVEL = 1e-05


def _fused_kernel(state_ref, w_ref, noise_ref, out_ref):
    q = jnp.dot(state_ref[...], w_ref[...], preferred_element_type=jnp.float32)
    q = q + noise_ref[...]
    j = jnp.argmax(q, axis=-1)
    a = q.shape[-1]
    iota = lax.broadcasted_iota(jnp.int32, q.shape, 1)
    out_ref[...] = (iota == j[:, None]).astype(jnp.float32)


@functools.partial(jax.jit, static_argnames=("tb",))
def _run(state, W, noise, tb):
    B, d = state.shape
    A = W.shape[1]
    grid = (B // tb,)
    return pl.pallas_call(
        _fused_kernel,
        grid=grid,
        in_specs=[
            pl.BlockSpec((tb, d), lambda i: (i, 0)),
            pl.BlockSpec((d, A), lambda i: (0, 0)),
            pl.BlockSpec((tb, A), lambda i: (i, 0)),
        ],
        out_specs=pl.BlockSpec((tb, A), lambda i: (i, 0)),
        out_shape=jax.ShapeDtypeStruct((B, A), jnp.float32),
    )(state, W, noise)


def kernel(state, W):
    B, d = state.shape
    A = W.shape[1]
    with jax.ensure_compile_time_eval():
        rand = jax.random.uniform(jax.random.key(1), (B, A), dtype=jnp.float32)
        noise = (rand * 2 - 1) * _NOISE_LEVEL
    tb = 256 if B % 256 == 0 else B
    return _run(state, W, noise, tb)


# speedup vs baseline: 1.8520x; 1.0193x over previous
import functools

import jax
import jax.numpy as jnp
from jax import lax
from jax.experimental import pallas as pl

_NOISE_LEVEL = 1e-05


def _fused_kernel(state_ref, w_ref, noise_ref, out_ref):
    q = jnp.dot(state_ref[...], w_ref[...], preferred_element_type=jnp.float32)
    q = q + noise_ref[...]
    j = jnp.argmax(q, axis=-1)
    a = q.shape[-1]
    iota = lax.broadcasted_iota(jnp.int32, q.shape, 1)
    out_ref[...] = (iota == j[:, None]).astype(jnp.float32)


@functools.partial(jax.jit, static_argnames=("tb",))
def _run(state, W, noise, tb):
    B, d = state.shape
    A = W.shape[1]
    grid = (B // tb,)
    return pl.pallas_call(
        _fused_kernel,
        grid=grid,
        in_specs=[
            pl.BlockSpec((tb, d), lambda i: (i, 0)),
            pl.BlockSpec((d, A), lambda i: (0, 0)),
            pl.BlockSpec((tb, A), lambda i: (i, 0)),
        ],
        out_specs=pl.BlockSpec((tb, A), lambda i: (i, 0)),
        out_shape=jax.ShapeDtypeStruct((B, A), jnp.float32),
    )(state, W, noise)


def kernel(state, W):
    B, d = state.shape
    A = W.shape[1]
    with jax.ensure_compile_time_eval():
        rand = jax.random.uniform(jax.random.key(1), (B, A), dtype=jnp.float32)
        noise = (rand * 2 - 1) * _NOISE_LEVEL
    tb = 512 if B % 512 == 0 else B
    return _run(state, W, noise, tb)
